# baseline (speedup 1.0000x reference)
import functools

import jax
import jax.numpy as jnp
from jax import lax
from jax.experimental import pallas as pl
from jax.experimental.pallas import tpu as pltpu
from jax.experimental.pallas import tpu_sc as plsc

_B = 1024
_S = 10
_K = 20
_H = 200
_LO = 128
_HI = 96
_HR = _H - _LO
_V = 10000
_NW = 32
_GROUPS = _B * _S
_GPW = _GROUPS // _NW
_G = 8
_NCH = _GPW // _G
_IPC = _G * _K
_IPW = _GPW * _K
_PPW = _B // _NW
_BV = 2048
_NVB = 5


def _split_body(entT_ref, relT_ref, el_ref, eh_ref, rl_ref, rh_ref):
    zeros = jnp.zeros((el_ref.shape[0], _HI - _HR), jnp.bfloat16)
    ent = entT_ref[...].T
    rel = relT_ref[...].T
    el_ref[...] = ent[:, :_LO].astype(jnp.bfloat16)
    eh_ref[:, :_HR] = ent[:, _LO:].astype(jnp.bfloat16)
    eh_ref[:, _HR:] = zeros
    rl_ref[...] = rel[:, :_LO].astype(jnp.bfloat16)
    rh_ref[:, :_HR] = rel[:, _LO:].astype(jnp.bfloat16)
    rh_ref[:, _HR:] = zeros


def _split_tables(entT, relT):
    return pl.pallas_call(
        _split_body,
        out_shape=[
            jax.ShapeDtypeStruct((_V, _LO), jnp.bfloat16),
            jax.ShapeDtypeStruct((_V, _HI), jnp.bfloat16),
            jax.ShapeDtypeStruct((_V, _LO), jnp.bfloat16),
            jax.ShapeDtypeStruct((_V, _HI), jnp.bfloat16),
        ],
    )(entT, relT)


def _accum_chunk(rows_v, acc_v, width):
    @pl.loop(0, _G)
    def _(g):
        for d in range(width // 32):
            sl = pl.ds(d * 32, 32)
            vals = [rows_v[g * _K + kk, sl] for kk in range(_K)]
            while len(vals) > 1:
                nxt = [a + b for a, b in zip(vals[::2], vals[1::2])]
                if len(vals) % 2:
                    nxt.append(vals[-1])
                vals = nxt
            acc_v[g, sl] = vals[0]


@functools.lru_cache(maxsize=None)
def _make_sc_branch(with_points):
    mesh = plsc.VectorSubcoreMesh(core_axis_name="c", subcore_axis_name="s")
    n_pts = 2 if with_points else 1
    out_type = [
        jax.ShapeDtypeStruct((_GROUPS, _LO), jnp.bfloat16),
        jax.ShapeDtypeStruct((_GROUPS, _LO), jnp.bfloat16),
    ] + [
        jax.ShapeDtypeStruct((_B, _LO), jnp.bfloat16),
        jax.ShapeDtypeStruct((_B, _HI), jnp.bfloat16),
    ] * n_pts
    scratch = [
        pltpu.VMEM((_IPW,), jnp.int32),
        pltpu.VMEM((_IPC, _LO), jnp.bfloat16),
        pltpu.VMEM((_IPC, _LO), jnp.bfloat16),
        pltpu.VMEM((_IPC, _HI), jnp.bfloat16),
        pltpu.VMEM((_IPC, _HI), jnp.bfloat16),
        pltpu.VMEM((_G, _LO), jnp.bfloat16),
        pltpu.VMEM((_G, _LO), jnp.bfloat16),
        pltpu.VMEM((_G, _LO), jnp.bfloat16),
        pltpu.VMEM((_G, _LO), jnp.bfloat16),
        pltpu.SemaphoreType.DMA,
        pltpu.SemaphoreType.DMA,
        pltpu.SemaphoreType.DMA,
        pltpu.SemaphoreType.DMA,
        pltpu.VMEM((_PPW,), jnp.int32),
        pltpu.VMEM((_PPW, _LO), jnp.bfloat16),
        pltpu.VMEM((_PPW, _HI), jnp.bfloat16),
    ]

    def body(tl_hbm, th_hbm, hist_hbm, pts_hbm, outs, pt_tables, scr):
        (idx_v, rlo0, rlo1, rhi0, rhi1, alo0, alo1, ahi0, ahi1,
         sg0, sg1, so0, so1, pidx_v, plo_v, phi_v) = scr
        slo_hbm, shi_hbm = outs[0], outs[1]
        wid = lax.axis_index("s") * 2 + lax.axis_index("c")
        base_g = wid * _GPW
        base_i = base_g * _K
        pltpu.sync_copy(hist_hbm.at[pl.ds(base_i, _IPW)], idx_v)

        def gathers(ci, rlo, rhi, sem):
            islice = idx_v.at[pl.ds(ci * _IPC, _IPC)]
            return (pltpu.make_async_copy(tl_hbm.at[islice], rlo, sem),
                    pltpu.make_async_copy(th_hbm.at[islice], rhi, sem))

        def outsd(ci, alo, ahi, sem):
            row = pl.ds(base_g + ci * _G, _G)
            return (pltpu.make_async_copy(alo, slo_hbm.at[row], sem),
                    pltpu.make_async_copy(ahi, shi_hbm.at[row], sem))

        def start(descs):
            for d in descs:
                d.start()

        def wait(descs):
            for d in descs:
                d.wait()

        start(gathers(0, rlo0, rhi0, sg0))

        @pl.loop(0, _NCH, step=2)
        def _(ci):
            start(gathers(ci + 1, rlo1, rhi1, sg1))
            wait(gathers(ci, rlo0, rhi0, sg0))

            @pl.when(ci >= 2)
            def _():
                wait(outsd(ci - 2, alo0, ahi0, so0))

            _accum_chunk(rlo0, alo0, _LO)
            _accum_chunk(rhi0, ahi0, _HI)
            start(outsd(ci, alo0, ahi0, so0))

            @pl.when(ci + 2 < _NCH)
            def _():
                start(gathers(ci + 2, rlo0, rhi0, sg0))

            wait(gathers(ci + 1, rlo1, rhi1, sg1))

            @pl.when(ci >= 2)
            def _():
                wait(outsd(ci - 1, alo1, ahi1, so1))

            _accum_chunk(rlo1, alo1, _LO)
            _accum_chunk(rhi1, ahi1, _HI)
            start(outsd(ci + 1, alo1, ahi1, so1))

        wait(outsd(_NCH - 2, alo0, ahi0, so0))
        wait(outsd(_NCH - 1, alo1, ahi1, so1))

        pb = wid * _PPW
        for j, (ptl, pth) in enumerate(pt_tables):
            prow = pl.ds(pb, _PPW)
            pltpu.sync_copy(pts_hbm.at[pl.ds(j * _B + pb, _PPW)], pidx_v)
            pltpu.async_copy(ptl.at[pidx_v], plo_v, sg0).wait()
            pltpu.async_copy(pth.at[pidx_v], phi_v, sg1).wait()
            pltpu.sync_copy(plo_v, outs[2 + 2 * j].at[prow])
            pltpu.sync_copy(phi_v, outs[3 + 2 * j].at[prow])

    if with_points:
        def k(el, eh, rl, rh, hist, pts, slo, shi, plo_s, phi_s,
              plo_r, phi_r, *scr):
            body(el, eh, hist, pts, (slo, shi, plo_s, phi_s, plo_r, phi_r),
                 [(el, eh), (rl, rh)], scr)
    else:
        def k(el, eh, hist, pts, dep, slo, shi, plo_o, phi_o, *scr):
            body(el, eh, hist, pts, (slo, shi, plo_o, phi_o),
                 [(el, eh)], scr)

    return functools.partial(
        pl.kernel, mesh=mesh, out_type=out_type,
        compiler_params=pltpu.CompilerParams(use_tc_tiling_on_sc=False),
        scratch_types=scratch,
    )(k)


def _dot_t(a, b):
    return lax.dot_general(a, b, (((1,), (1,)), ((), ())),
                           preferred_element_type=jnp.float32)


def _f32(x):
    return x.astype(jnp.float32)


def _xw(xl_ref, xh_ref, w_ref, c0):
    return (_dot_t(_f32(xl_ref[...]), w_ref[:, c0:c0 + _LO])
            + _dot_t(_f32(xh_ref[:, :_HR]), w_ref[:, c0 + _LO:c0 + _H]))


def _gru_body(sl_ref, sh_ref, el_ref, eh_ref, rl_ref, rh_ref,
              wih_ref, whh_ref, bih_ref, bhh_ref, h_ref):
    base = (_xw(el_ref, eh_ref, wih_ref, _H)
            + _xw(rl_ref, rh_ref, wih_ref, 2 * _H) + bih_ref[...])
    sc = 1.0 / _K
    wml = wih_ref[:, 0:_LO] * sc
    wmh = wih_ref[:, _LO:_H] * sc

    h = jnp.zeros((_B, _H), jnp.float32)
    for t in range(_S):
        row = slice(t * _B, (t + 1) * _B)
        gi = (_dot_t(_f32(sl_ref[row, :]), wml)
              + _dot_t(_f32(sh_ref[row, :_HR]), wmh) + base)
        gh = _dot_t(h, whh_ref[...]) + bhh_ref[...]
        rg = jax.nn.sigmoid(gi[:, 0:_H] + gh[:, 0:_H])
        zg = jax.nn.sigmoid(gi[:, _H:2 * _H] + gh[:, _H:2 * _H])
        ng = jnp.tanh(gi[:, 2 * _H:] + rg * gh[:, 2 * _H:])
        h = (1.0 - zg) * ng + zg * h
    h_ref[...] = h


def _gru(sl, sh, el, eh, rl, rh, W_ih, W_hh, b_ih, b_hh):
    return pl.pallas_call(
        _gru_body,
        out_shape=jax.ShapeDtypeStruct((_B, _H), jnp.float32),
    )(sl, sh, el, eh, rl, rh, W_ih, W_hh,
      b_ih.reshape(1, -1), b_hh.reshape(1, -1))


def _dot0(w, x):
    return lax.dot_general(w, x, (((0,), (1,)), ((), ())),
                           preferred_element_type=jnp.float32)


def _logits_body(el_ref, eh_ref, h_ref, rl_ref, rh_ref, wt_ref, b_ref,
                 tgt_ref, out_ref, loss_ref, m_s, s_s, t_s):
    i = pl.program_id(0)
    h16 = h_ref[...].astype(jnp.bfloat16)
    logits = (_dot0(wt_ref[0:_LO, :], el_ref[...])
              + _dot0(wt_ref[_LO:_H, :], eh_ref[:, :_HR])
              + _dot0(wt_ref[_H:2 * _H, :], h16)
              + _dot0(wt_ref[2 * _H:2 * _H + _LO, :], rl_ref[...])
              + _dot0(wt_ref[2 * _H + _LO:3 * _H, :], rh_ref[:, :_HR])
              + b_ref[...])
    out_ref[...] = logits
    col = i * _BV + lax.broadcasted_iota(jnp.int32, (_BV, 1), 0)
    lg = jnp.where(col < _V, logits, -1e30)
    bm = jnp.max(lg, axis=0, keepdims=True)
    tc = jnp.sum(jnp.where(col == tgt_ref[...], lg, 0.0), axis=0,
                 keepdims=True)

    @pl.when(i == 0)
    def _():
        m_s[...] = bm
        s_s[...] = jnp.sum(jnp.exp(lg - bm), axis=0, keepdims=True)
        t_s[...] = tc

    @pl.when(i > 0)
    def _():
        m_new = jnp.maximum(m_s[...], bm)
        s_s[...] = (s_s[...] * jnp.exp(m_s[...] - m_new)
                    + jnp.sum(jnp.exp(lg - m_new), axis=0, keepdims=True))
        m_s[...] = m_new
        t_s[...] = t_s[...] + tc

    @pl.when(i == _NVB - 1)
    def _():
        loss_ref[...] = jnp.sum(jnp.log(s_s[...]) + m_s[...] - t_s[...],
                                axis=1, keepdims=True) * (1.0 / _B)


def _logits_ce(el, eh, h, rl, rh, Wt, bcol, tgt_row):
    full = lambda shape: pl.BlockSpec(shape, lambda i: (0, 0))
    out_t, loss = pl.pallas_call(
        _logits_body,
        grid=(_NVB,),
        in_specs=[
            full((_B, _LO)),
            full((_B, _HI)),
            full((_B, _H)),
            full((_B, _LO)),
            full((_B, _HI)),
            pl.BlockSpec((3 * _H, _BV), lambda i: (0, i)),
            pl.BlockSpec((_BV, 1), lambda i: (i, 0)),
            full((1, _B)),
        ],
        out_specs=[
            pl.BlockSpec((_BV, _B), lambda i: (i, 0)),
            pl.BlockSpec((1, 1), lambda i: (0, 0)),
        ],
        out_shape=[
            jax.ShapeDtypeStruct((_V, _B), jnp.float32),
            jax.ShapeDtypeStruct((1, 1), jnp.float32),
        ],
        scratch_shapes=[
            pltpu.VMEM((1, _B), jnp.float32),
            pltpu.VMEM((1, _B), jnp.float32),
            pltpu.VMEM((1, _B), jnp.float32),
        ],
    )(el, eh, h, rl, rh, Wt, bcol, tgt_row)
    return out_t.T, loss


def kernel(triplets, s_hist, o_hist, ent_embeds, rel_embeds,
           W_ih_s, W_hh_s, b_ih_s, b_hh_s, W_ih_o, W_hh_o, b_ih_o, b_hh_o,
           W_sub, b_sub, W_ob, b_ob):
    s = triplets[:, 0].astype(jnp.int32)
    r = triplets[:, 1].astype(jnp.int32)
    o = triplets[:, 2].astype(jnp.int32)

    ent_lo, ent_hi, rel_lo, rel_hi = _split_tables(ent_embeds.T,
                                                   rel_embeds.T)
    s_idx = s_hist.transpose(1, 0, 2).reshape(-1).astype(jnp.int32)
    o_idx = o_hist.transpose(1, 0, 2).reshape(-1).astype(jnp.int32)
    pts_sr = jnp.concatenate([s, r])

    ssl, ssh, esl, esh, rrl, rrh = _make_sc_branch(True)(
        ent_lo, ent_hi, rel_lo, rel_hi, s_idx, pts_sr)
    osl, osh, eol, eoh = _make_sc_branch(False)(
        ent_lo, ent_hi, o_idx, o, ssl)

    s_h = _gru(ssl, ssh, esl, esh, rrl, rrh, W_ih_s, W_hh_s, b_ih_s, b_hh_s)
    o_h = _gru(osl, osh, eol, eoh, rrl, rrh, W_ih_o, W_hh_o, b_ih_o, b_hh_o)

    ob_pred, loss_ob = _logits_ce(esl, esh, s_h, rrl, rrh,
                                  W_sub.T.astype(jnp.bfloat16),
                                  b_sub.reshape(-1, 1), o.reshape(1, -1))
    sub_pred, loss_sub = _logits_ce(eol, eoh, o_h, rrl, rrh,
                                    W_ob.T.astype(jnp.bfloat16),
                                    b_ob.reshape(-1, 1), s.reshape(1, -1))

    loss = (loss_ob + loss_sub).reshape(())
    return (loss, sub_pred, ob_pred)


# speedup vs baseline: 1.0928x; 1.0928x over previous
import functools

import jax
import jax.numpy as jnp
from jax import lax
from jax.experimental import pallas as pl
from jax.experimental.pallas import tpu as pltpu
from jax.experimental.pallas import tpu_sc as plsc

_B = 1024
_S = 10
_K = 20
_H = 200
_LO = 128
_HI = 96
_HR = _H - _LO
_V = 10000
_NW = 32
_GROUPS = _B * _S
_GPW = _GROUPS // _NW
_G = 8
_NCH = _GPW // _G
_IPC = _G * _K
_IPW = _GPW * _K
_PPW = _B // _NW
_BV = 2048
_NVB = 5


def _split_body(entT_ref, relT_ref, el_ref, eh_ref, rl_ref, rh_ref):
    zeros = jnp.zeros((el_ref.shape[0], _HI - _HR), jnp.bfloat16)
    ent = entT_ref[...].T
    rel = relT_ref[...].T
    el_ref[...] = ent[:, :_LO].astype(jnp.bfloat16)
    eh_ref[:, :_HR] = ent[:, _LO:].astype(jnp.bfloat16)
    eh_ref[:, _HR:] = zeros
    rl_ref[...] = rel[:, :_LO].astype(jnp.bfloat16)
    rh_ref[:, :_HR] = rel[:, _LO:].astype(jnp.bfloat16)
    rh_ref[:, _HR:] = zeros


def _split_tables(entT, relT):
    return pl.pallas_call(
        _split_body,
        out_shape=[
            jax.ShapeDtypeStruct((_V, _LO), jnp.bfloat16),
            jax.ShapeDtypeStruct((_V, _HI), jnp.bfloat16),
            jax.ShapeDtypeStruct((_V, _LO), jnp.bfloat16),
            jax.ShapeDtypeStruct((_V, _HI), jnp.bfloat16),
        ],
    )(entT, relT)


def _accum_chunk(rows_v, acc_v, width):
    @pl.loop(0, _G)
    def _(g):
        for d in range(width // 32):
            sl = pl.ds(d * 32, 32)
            vals = [rows_v[g * _K + kk, sl] for kk in range(_K)]
            while len(vals) > 1:
                nxt = [a + b for a, b in zip(vals[::2], vals[1::2])]
                if len(vals) % 2:
                    nxt.append(vals[-1])
                vals = nxt
            acc_v[g, sl] = vals[0]


@functools.lru_cache(maxsize=None)
def _make_sc_branch(with_points):
    mesh = plsc.VectorSubcoreMesh(core_axis_name="c", subcore_axis_name="s")
    n_pts = 2 if with_points else 1
    out_type = [
        jax.ShapeDtypeStruct((_GROUPS, _LO), jnp.bfloat16),
        jax.ShapeDtypeStruct((_GROUPS, _LO), jnp.bfloat16),
    ] + [
        jax.ShapeDtypeStruct((_B, _LO), jnp.bfloat16),
        jax.ShapeDtypeStruct((_B, _HI), jnp.bfloat16),
    ] * n_pts
    scratch = [
        pltpu.VMEM((_IPW,), jnp.int32),
        pltpu.VMEM((_IPC, _LO), jnp.bfloat16),
        pltpu.VMEM((_IPC, _LO), jnp.bfloat16),
        pltpu.VMEM((_IPC, _HI), jnp.bfloat16),
        pltpu.VMEM((_IPC, _HI), jnp.bfloat16),
        pltpu.VMEM((_G, _LO), jnp.bfloat16),
        pltpu.VMEM((_G, _LO), jnp.bfloat16),
        pltpu.VMEM((_G, _LO), jnp.bfloat16),
        pltpu.VMEM((_G, _LO), jnp.bfloat16),
        pltpu.SemaphoreType.DMA,
        pltpu.SemaphoreType.DMA,
        pltpu.SemaphoreType.DMA,
        pltpu.SemaphoreType.DMA,
        pltpu.VMEM((_PPW,), jnp.int32),
        pltpu.VMEM((_PPW, _LO), jnp.bfloat16),
        pltpu.VMEM((_PPW, _HI), jnp.bfloat16),
    ]

    def body(tl_hbm, th_hbm, hist_hbm, pts_hbm, outs, pt_tables, scr):
        (idx_v, rlo0, rlo1, rhi0, rhi1, alo0, alo1, ahi0, ahi1,
         sg0, sg1, so0, so1, pidx_v, plo_v, phi_v) = scr
        slo_hbm, shi_hbm = outs[0], outs[1]
        wid = lax.axis_index("s") * 2 + lax.axis_index("c")
        base_g = wid * _GPW
        base_i = base_g * _K
        pltpu.sync_copy(hist_hbm.at[pl.ds(base_i, _IPW)], idx_v)

        def gathers(ci, rlo, rhi, sem):
            islice = idx_v.at[pl.ds(ci * _IPC, _IPC)]
            return (pltpu.make_async_copy(tl_hbm.at[islice], rlo, sem),
                    pltpu.make_async_copy(th_hbm.at[islice], rhi, sem))

        def outsd(ci, alo, ahi, sem):
            row = pl.ds(base_g + ci * _G, _G)
            return (pltpu.make_async_copy(alo, slo_hbm.at[row], sem),
                    pltpu.make_async_copy(ahi, shi_hbm.at[row], sem))

        def start(descs):
            for d in descs:
                d.start()

        def wait(descs):
            for d in descs:
                d.wait()

        start(gathers(0, rlo0, rhi0, sg0))

        @pl.loop(0, _NCH, step=2)
        def _(ci):
            start(gathers(ci + 1, rlo1, rhi1, sg1))
            wait(gathers(ci, rlo0, rhi0, sg0))

            @pl.when(ci >= 2)
            def _():
                wait(outsd(ci - 2, alo0, ahi0, so0))

            _accum_chunk(rlo0, alo0, _LO)
            _accum_chunk(rhi0, ahi0, _HI)
            start(outsd(ci, alo0, ahi0, so0))

            @pl.when(ci + 2 < _NCH)
            def _():
                start(gathers(ci + 2, rlo0, rhi0, sg0))

            wait(gathers(ci + 1, rlo1, rhi1, sg1))

            @pl.when(ci >= 2)
            def _():
                wait(outsd(ci - 1, alo1, ahi1, so1))

            _accum_chunk(rlo1, alo1, _LO)
            _accum_chunk(rhi1, ahi1, _HI)
            start(outsd(ci + 1, alo1, ahi1, so1))

        wait(outsd(_NCH - 2, alo0, ahi0, so0))
        wait(outsd(_NCH - 1, alo1, ahi1, so1))

        pb = wid * _PPW
        for j, (ptl, pth) in enumerate(pt_tables):
            prow = pl.ds(pb, _PPW)
            pltpu.sync_copy(pts_hbm.at[pl.ds(j * _B + pb, _PPW)], pidx_v)
            pltpu.async_copy(ptl.at[pidx_v], plo_v, sg0).wait()
            pltpu.async_copy(pth.at[pidx_v], phi_v, sg1).wait()
            pltpu.sync_copy(plo_v, outs[2 + 2 * j].at[prow])
            pltpu.sync_copy(phi_v, outs[3 + 2 * j].at[prow])

    if with_points:
        def k(el, eh, rl, rh, hist, pts, slo, shi, plo_s, phi_s,
              plo_r, phi_r, *scr):
            body(el, eh, hist, pts, (slo, shi, plo_s, phi_s, plo_r, phi_r),
                 [(el, eh), (rl, rh)], scr)
    else:
        def k(el, eh, hist, pts, dep, slo, shi, plo_o, phi_o, *scr):
            body(el, eh, hist, pts, (slo, shi, plo_o, phi_o),
                 [(el, eh)], scr)

    return functools.partial(
        pl.kernel, mesh=mesh, out_type=out_type,
        compiler_params=pltpu.CompilerParams(use_tc_tiling_on_sc=False),
        scratch_types=scratch,
    )(k)


def _dot_t(a, b):
    return lax.dot_general(a, b, (((1,), (1,)), ((), ())),
                           preferred_element_type=jnp.float32)


def _f32(x):
    return x.astype(jnp.float32)


def _xw(xl_ref, xh_ref, w_ref, c0):
    return (_dot_t(_f32(xl_ref[...]), w_ref[:, c0:c0 + _LO])
            + _dot_t(_f32(xh_ref[:, :_HR]), w_ref[:, c0 + _LO:c0 + _H]))


def _gru_body(sl_ref, sh_ref, el_ref, eh_ref, rl_ref, rh_ref,
              wih_ref, whh_ref, bih_ref, bhh_ref, h_ref):
    base = (_xw(el_ref, eh_ref, wih_ref, _H)
            + _xw(rl_ref, rh_ref, wih_ref, 2 * _H) + bih_ref[...])
    sc = 1.0 / _K
    wml = wih_ref[:, 0:_LO] * sc
    wmh = wih_ref[:, _LO:_H] * sc

    h = jnp.zeros((_B, _H), jnp.float32)
    for t in range(_S):
        row = slice(t * _B, (t + 1) * _B)
        gi = (_dot_t(_f32(sl_ref[row, :]), wml)
              + _dot_t(_f32(sh_ref[row, :_HR]), wmh) + base)
        gh = _dot_t(h, whh_ref[...]) + bhh_ref[...]
        rg = jax.nn.sigmoid(gi[:, 0:_H] + gh[:, 0:_H])
        zg = jax.nn.sigmoid(gi[:, _H:2 * _H] + gh[:, _H:2 * _H])
        ng = jnp.tanh(gi[:, 2 * _H:] + rg * gh[:, 2 * _H:])
        h = (1.0 - zg) * ng + zg * h
    h_ref[...] = h


def _gru(sl, sh, el, eh, rl, rh, W_ih, W_hh, b_ih, b_hh):
    return pl.pallas_call(
        _gru_body,
        out_shape=jax.ShapeDtypeStruct((_B, _H), jnp.float32),
    )(sl, sh, el, eh, rl, rh, W_ih, W_hh,
      b_ih.reshape(1, -1), b_hh.reshape(1, -1))


def _dot0(w, x):
    return lax.dot_general(w.astype(jnp.bfloat16), x, (((0,), (1,)), ((), ())),
                           preferred_element_type=jnp.float32)


def _logits_body(el_ref, eh_ref, h_ref, rl_ref, rh_ref, wt_ref, b_ref,
                 tgt_ref, out_ref, loss_ref, m_s, s_s, t_s):
    i = pl.program_id(0)
    h16 = h_ref[...].astype(jnp.bfloat16)
    logits = (_dot0(wt_ref[0:_LO, :], el_ref[...])
              + _dot0(wt_ref[_LO:_H, :], eh_ref[:, :_HR])
              + _dot0(wt_ref[_H:2 * _H, :], h16)
              + _dot0(wt_ref[2 * _H:2 * _H + _LO, :], rl_ref[...])
              + _dot0(wt_ref[2 * _H + _LO:3 * _H, :], rh_ref[:, :_HR])
              + b_ref[...])
    out_ref[...] = logits
    col = i * _BV + lax.broadcasted_iota(jnp.int32, (_BV, 1), 0)
    lg = jnp.where(col < _V, logits, -1e30)
    bm = jnp.max(lg, axis=0, keepdims=True)
    tc = jnp.sum(jnp.where(col == tgt_ref[...], lg, 0.0), axis=0,
                 keepdims=True)

    @pl.when(i == 0)
    def _():
        m_s[...] = bm
        s_s[...] = jnp.sum(jnp.exp(lg - bm), axis=0, keepdims=True)
        t_s[...] = tc

    @pl.when(i > 0)
    def _():
        m_new = jnp.maximum(m_s[...], bm)
        s_s[...] = (s_s[...] * jnp.exp(m_s[...] - m_new)
                    + jnp.sum(jnp.exp(lg - m_new), axis=0, keepdims=True))
        m_s[...] = m_new
        t_s[...] = t_s[...] + tc

    @pl.when(i == _NVB - 1)
    def _():
        loss_ref[...] = jnp.sum(jnp.log(s_s[...]) + m_s[...] - t_s[...],
                                axis=1, keepdims=True) * (1.0 / _B)


def _logits_ce(el, eh, h, rl, rh, Wt, bcol, tgt_row):
    full = lambda shape: pl.BlockSpec(shape, lambda i: (0, 0))
    out_t, loss = pl.pallas_call(
        _logits_body,
        grid=(_NVB,),
        in_specs=[
            full((_B, _LO)),
            full((_B, _HI)),
            full((_B, _H)),
            full((_B, _LO)),
            full((_B, _HI)),
            pl.BlockSpec((3 * _H, _BV), lambda i: (0, i)),
            pl.BlockSpec((_BV, 1), lambda i: (i, 0)),
            full((1, _B)),
        ],
        out_specs=[
            pl.BlockSpec((_BV, _B), lambda i: (i, 0)),
            pl.BlockSpec((1, 1), lambda i: (0, 0)),
        ],
        out_shape=[
            jax.ShapeDtypeStruct((_V, _B), jnp.float32),
            jax.ShapeDtypeStruct((1, 1), jnp.float32),
        ],
        scratch_shapes=[
            pltpu.VMEM((1, _B), jnp.float32),
            pltpu.VMEM((1, _B), jnp.float32),
            pltpu.VMEM((1, _B), jnp.float32),
        ],
    )(el, eh, h, rl, rh, Wt, bcol, tgt_row)
    return out_t.T, loss


def kernel(triplets, s_hist, o_hist, ent_embeds, rel_embeds,
           W_ih_s, W_hh_s, b_ih_s, b_hh_s, W_ih_o, W_hh_o, b_ih_o, b_hh_o,
           W_sub, b_sub, W_ob, b_ob):
    s = triplets[:, 0].astype(jnp.int32)
    r = triplets[:, 1].astype(jnp.int32)
    o = triplets[:, 2].astype(jnp.int32)

    ent_lo, ent_hi, rel_lo, rel_hi = _split_tables(ent_embeds.T,
                                                   rel_embeds.T)
    s_idx = s_hist.transpose(1, 0, 2).reshape(-1).astype(jnp.int32)
    o_idx = o_hist.transpose(1, 0, 2).reshape(-1).astype(jnp.int32)
    pts_sr = jnp.concatenate([s, r])

    ssl, ssh, esl, esh, rrl, rrh = _make_sc_branch(True)(
        ent_lo, ent_hi, rel_lo, rel_hi, s_idx, pts_sr)
    osl, osh, eol, eoh = _make_sc_branch(False)(
        ent_lo, ent_hi, o_idx, o, ssl)

    s_h = _gru(ssl, ssh, esl, esh, rrl, rrh, W_ih_s, W_hh_s, b_ih_s, b_hh_s)
    o_h = _gru(osl, osh, eol, eoh, rrl, rrh, W_ih_o, W_hh_o, b_ih_o, b_hh_o)

    ob_pred, loss_ob = _logits_ce(esl, esh, s_h, rrl, rrh, W_sub.T,
                                  b_sub.reshape(-1, 1), o.reshape(1, -1))
    sub_pred, loss_sub = _logits_ce(eol, eoh, o_h, rrl, rrh, W_ob.T,
                                    b_ob.reshape(-1, 1), s.reshape(1, -1))

    loss = (loss_ob + loss_sub).reshape(())
    return (loss, sub_pred, ob_pred)


# speedup vs baseline: 1.0955x; 1.0024x over previous
import functools

import jax
import jax.numpy as jnp
from jax import lax
from jax.experimental import pallas as pl
from jax.experimental.pallas import tpu as pltpu
from jax.experimental.pallas import tpu_sc as plsc

_B = 1024
_S = 10
_K = 20
_H = 200
_LO = 128
_HI = 96
_HR = _H - _LO
_V = 10000
_NW = 32
_GROUPS = _B * _S
_GPW = _GROUPS // _NW
_G = 8
_NCH = _GPW // _G
_IPC = _G * _K
_IPW = _GPW * _K
_PPW = _B // _NW
_BV = 2048
_NVB = 5


def _split_body(entT_ref, relT_ref, el_ref, eh_ref, rl_ref, rh_ref):
    zeros = jnp.zeros((el_ref.shape[0], _HI - _HR), jnp.bfloat16)
    ent = entT_ref[...].T
    rel = relT_ref[...].T
    el_ref[...] = ent[:, :_LO].astype(jnp.bfloat16)
    eh_ref[:, :_HR] = ent[:, _LO:].astype(jnp.bfloat16)
    eh_ref[:, _HR:] = zeros
    rl_ref[...] = rel[:, :_LO].astype(jnp.bfloat16)
    rh_ref[:, :_HR] = rel[:, _LO:].astype(jnp.bfloat16)
    rh_ref[:, _HR:] = zeros


def _split_tables(entT, relT):
    return pl.pallas_call(
        _split_body,
        out_shape=[
            jax.ShapeDtypeStruct((_V, _LO), jnp.bfloat16),
            jax.ShapeDtypeStruct((_V, _HI), jnp.bfloat16),
            jax.ShapeDtypeStruct((_V, _LO), jnp.bfloat16),
            jax.ShapeDtypeStruct((_V, _HI), jnp.bfloat16),
        ],
    )(entT, relT)


def _accum_chunk(rows_v, acc_v, width):
    @pl.loop(0, _G)
    def _(g):
        for d in range(width // 32):
            sl = pl.ds(d * 32, 32)
            vals = [rows_v[g * _K + kk, sl] for kk in range(_K)]
            while len(vals) > 1:
                nxt = [a + b for a, b in zip(vals[::2], vals[1::2])]
                if len(vals) % 2:
                    nxt.append(vals[-1])
                vals = nxt
            acc_v[g, sl] = vals[0]


@functools.lru_cache(maxsize=None)
def _make_sc_branch(with_points):
    mesh = plsc.VectorSubcoreMesh(core_axis_name="c", subcore_axis_name="s")
    n_pts = 2 if with_points else 1
    out_type = [
        jax.ShapeDtypeStruct((_GROUPS, _LO), jnp.bfloat16),
        jax.ShapeDtypeStruct((_GROUPS, _LO), jnp.bfloat16),
    ] + [
        jax.ShapeDtypeStruct((_B, _LO), jnp.bfloat16),
        jax.ShapeDtypeStruct((_B, _HI), jnp.bfloat16),
    ] * n_pts
    scratch = [
        pltpu.VMEM((_IPW,), jnp.int32),
        pltpu.VMEM((_IPC, _LO), jnp.bfloat16),
        pltpu.VMEM((_IPC, _LO), jnp.bfloat16),
        pltpu.VMEM((_IPC, _HI), jnp.bfloat16),
        pltpu.VMEM((_IPC, _HI), jnp.bfloat16),
        pltpu.VMEM((_G, _LO), jnp.bfloat16),
        pltpu.VMEM((_G, _LO), jnp.bfloat16),
        pltpu.VMEM((_G, _LO), jnp.bfloat16),
        pltpu.VMEM((_G, _LO), jnp.bfloat16),
        pltpu.SemaphoreType.DMA,
        pltpu.SemaphoreType.DMA,
        pltpu.SemaphoreType.DMA,
        pltpu.SemaphoreType.DMA,
        pltpu.VMEM((_PPW,), jnp.int32),
        pltpu.VMEM((_PPW, _LO), jnp.bfloat16),
        pltpu.VMEM((_PPW, _HI), jnp.bfloat16),
    ]

    def body(tl_hbm, th_hbm, hist_hbm, pts_hbm, outs, pt_tables, scr):
        (idx_v, rlo0, rlo1, rhi0, rhi1, alo0, alo1, ahi0, ahi1,
         sg0, sg1, so0, so1, pidx_v, plo_v, phi_v) = scr
        slo_hbm, shi_hbm = outs[0], outs[1]
        wid = lax.axis_index("s") * 2 + lax.axis_index("c")
        base_g = wid * _GPW
        base_i = base_g * _K
        pltpu.sync_copy(hist_hbm.at[pl.ds(base_i, _IPW)], idx_v)

        def gathers(ci, rlo, rhi, sem):
            islice = idx_v.at[pl.ds(ci * _IPC, _IPC)]
            return (pltpu.make_async_copy(tl_hbm.at[islice], rlo, sem),
                    pltpu.make_async_copy(th_hbm.at[islice], rhi, sem))

        def outsd(ci, alo, ahi, sem):
            row = pl.ds(base_g + ci * _G, _G)
            return (pltpu.make_async_copy(alo, slo_hbm.at[row], sem),
                    pltpu.make_async_copy(ahi, shi_hbm.at[row], sem))

        def start(descs):
            for d in descs:
                d.start()

        def wait(descs):
            for d in descs:
                d.wait()

        start(gathers(0, rlo0, rhi0, sg0))

        @pl.loop(0, _NCH, step=2)
        def _(ci):
            start(gathers(ci + 1, rlo1, rhi1, sg1))
            wait(gathers(ci, rlo0, rhi0, sg0))

            @pl.when(ci >= 2)
            def _():
                wait(outsd(ci - 2, alo0, ahi0, so0))

            _accum_chunk(rlo0, alo0, _LO)
            _accum_chunk(rhi0, ahi0, _HI)
            start(outsd(ci, alo0, ahi0, so0))

            @pl.when(ci + 2 < _NCH)
            def _():
                start(gathers(ci + 2, rlo0, rhi0, sg0))

            wait(gathers(ci + 1, rlo1, rhi1, sg1))

            @pl.when(ci >= 2)
            def _():
                wait(outsd(ci - 1, alo1, ahi1, so1))

            _accum_chunk(rlo1, alo1, _LO)
            _accum_chunk(rhi1, ahi1, _HI)
            start(outsd(ci + 1, alo1, ahi1, so1))

        wait(outsd(_NCH - 2, alo0, ahi0, so0))
        wait(outsd(_NCH - 1, alo1, ahi1, so1))

        pb = wid * _PPW
        for j, (ptl, pth) in enumerate(pt_tables):
            prow = pl.ds(pb, _PPW)
            pltpu.sync_copy(pts_hbm.at[pl.ds(j * _B + pb, _PPW)], pidx_v)
            pltpu.async_copy(ptl.at[pidx_v], plo_v, sg0).wait()
            pltpu.async_copy(pth.at[pidx_v], phi_v, sg1).wait()
            pltpu.sync_copy(plo_v, outs[2 + 2 * j].at[prow])
            pltpu.sync_copy(phi_v, outs[3 + 2 * j].at[prow])

    if with_points:
        def k(el, eh, rl, rh, hist, pts, slo, shi, plo_s, phi_s,
              plo_r, phi_r, *scr):
            body(el, eh, hist, pts, (slo, shi, plo_s, phi_s, plo_r, phi_r),
                 [(el, eh), (rl, rh)], scr)
    else:
        def k(el, eh, hist, pts, dep, slo, shi, plo_o, phi_o, *scr):
            body(el, eh, hist, pts, (slo, shi, plo_o, phi_o),
                 [(el, eh)], scr)

    return functools.partial(
        pl.kernel, mesh=mesh, out_type=out_type,
        compiler_params=pltpu.CompilerParams(use_tc_tiling_on_sc=False),
        scratch_types=scratch,
    )(k)


def _dot_t(a, b):
    return lax.dot_general(a, b, (((1,), (1,)), ((), ())),
                           preferred_element_type=jnp.float32)


def _f32(x):
    return x.astype(jnp.float32)


def _bf(x):
    return x.astype(jnp.bfloat16)


def _xw(xl_ref, xh_ref, w_ref, c0):
    return (_dot_t(xl_ref[...], _bf(w_ref[:, c0:c0 + _LO]))
            + _dot_t(xh_ref[:, :_HR], _bf(w_ref[:, c0 + _LO:c0 + _H])))


def _gru_body(sl_ref, sh_ref, el_ref, eh_ref, rl_ref, rh_ref,
              wih_ref, whh_ref, bih_ref, bhh_ref, h_ref):
    base = (_xw(el_ref, eh_ref, wih_ref, _H)
            + _xw(rl_ref, rh_ref, wih_ref, 2 * _H) + bih_ref[...])
    sc = 1.0 / _K
    wml = _bf(wih_ref[:, 0:_LO] * sc)
    wmh = _bf(wih_ref[:, _LO:_H] * sc)
    whh = _bf(whh_ref[...])

    h = jnp.zeros((_B, _H), jnp.float32)
    for t in range(_S):
        row = slice(t * _B, (t + 1) * _B)
        gi = (_dot_t(sl_ref[row, :], wml)
              + _dot_t(sh_ref[row, :_HR], wmh) + base)
        gh = _dot_t(_bf(h), whh) + bhh_ref[...]
        rg = jax.nn.sigmoid(gi[:, 0:_H] + gh[:, 0:_H])
        zg = jax.nn.sigmoid(gi[:, _H:2 * _H] + gh[:, _H:2 * _H])
        ng = jnp.tanh(gi[:, 2 * _H:] + rg * gh[:, 2 * _H:])
        h = (1.0 - zg) * ng + zg * h
    h_ref[...] = h


def _gru(sl, sh, el, eh, rl, rh, W_ih, W_hh, b_ih, b_hh):
    return pl.pallas_call(
        _gru_body,
        out_shape=jax.ShapeDtypeStruct((_B, _H), jnp.float32),
    )(sl, sh, el, eh, rl, rh, W_ih, W_hh,
      b_ih.reshape(1, -1), b_hh.reshape(1, -1))


def _dot0(w, x):
    return lax.dot_general(w.astype(jnp.bfloat16), x, (((0,), (1,)), ((), ())),
                           preferred_element_type=jnp.float32)


def _logits_body(el_ref, eh_ref, h_ref, rl_ref, rh_ref, wt_ref, b_ref,
                 tgt_ref, out_ref, loss_ref, m_s, s_s, t_s):
    i = pl.program_id(0)
    h16 = h_ref[...].astype(jnp.bfloat16)
    logits = (_dot0(wt_ref[0:_LO, :], el_ref[...])
              + _dot0(wt_ref[_LO:_H, :], eh_ref[:, :_HR])
              + _dot0(wt_ref[_H:2 * _H, :], h16)
              + _dot0(wt_ref[2 * _H:2 * _H + _LO, :], rl_ref[...])
              + _dot0(wt_ref[2 * _H + _LO:3 * _H, :], rh_ref[:, :_HR])
              + b_ref[...])
    out_ref[...] = logits
    col = i * _BV + lax.broadcasted_iota(jnp.int32, (_BV, 1), 0)
    lg = jnp.where(col < _V, logits, -1e30)
    bm = jnp.max(lg, axis=0, keepdims=True)
    tc = jnp.sum(jnp.where(col == tgt_ref[...], lg, 0.0), axis=0,
                 keepdims=True)

    @pl.when(i == 0)
    def _():
        m_s[...] = bm
        s_s[...] = jnp.sum(jnp.exp(lg - bm), axis=0, keepdims=True)
        t_s[...] = tc

    @pl.when(i > 0)
    def _():
        m_new = jnp.maximum(m_s[...], bm)
        s_s[...] = (s_s[...] * jnp.exp(m_s[...] - m_new)
                    + jnp.sum(jnp.exp(lg - m_new), axis=0, keepdims=True))
        m_s[...] = m_new
        t_s[...] = t_s[...] + tc

    @pl.when(i == _NVB - 1)
    def _():
        loss_ref[...] = jnp.sum(jnp.log(s_s[...]) + m_s[...] - t_s[...],
                                axis=1, keepdims=True) * (1.0 / _B)


def _logits_ce(el, eh, h, rl, rh, Wt, bcol, tgt_row):
    full = lambda shape: pl.BlockSpec(shape, lambda i: (0, 0))
    out_t, loss = pl.pallas_call(
        _logits_body,
        grid=(_NVB,),
        in_specs=[
            full((_B, _LO)),
            full((_B, _HI)),
            full((_B, _H)),
            full((_B, _LO)),
            full((_B, _HI)),
            pl.BlockSpec((3 * _H, _BV), lambda i: (0, i)),
            pl.BlockSpec((_BV, 1), lambda i: (i, 0)),
            full((1, _B)),
        ],
        out_specs=[
            pl.BlockSpec((_BV, _B), lambda i: (i, 0)),
            pl.BlockSpec((1, 1), lambda i: (0, 0)),
        ],
        out_shape=[
            jax.ShapeDtypeStruct((_V, _B), jnp.float32),
            jax.ShapeDtypeStruct((1, 1), jnp.float32),
        ],
        scratch_shapes=[
            pltpu.VMEM((1, _B), jnp.float32),
            pltpu.VMEM((1, _B), jnp.float32),
            pltpu.VMEM((1, _B), jnp.float32),
        ],
    )(el, eh, h, rl, rh, Wt, bcol, tgt_row)
    return out_t.T, loss


def kernel(triplets, s_hist, o_hist, ent_embeds, rel_embeds,
           W_ih_s, W_hh_s, b_ih_s, b_hh_s, W_ih_o, W_hh_o, b_ih_o, b_hh_o,
           W_sub, b_sub, W_ob, b_ob):
    s = triplets[:, 0].astype(jnp.int32)
    r = triplets[:, 1].astype(jnp.int32)
    o = triplets[:, 2].astype(jnp.int32)

    ent_lo, ent_hi, rel_lo, rel_hi = _split_tables(ent_embeds.T,
                                                   rel_embeds.T)
    s_idx = s_hist.transpose(1, 0, 2).reshape(-1).astype(jnp.int32)
    o_idx = o_hist.transpose(1, 0, 2).reshape(-1).astype(jnp.int32)
    pts_sr = jnp.concatenate([s, r])

    ssl, ssh, esl, esh, rrl, rrh = _make_sc_branch(True)(
        ent_lo, ent_hi, rel_lo, rel_hi, s_idx, pts_sr)
    osl, osh, eol, eoh = _make_sc_branch(False)(
        ent_lo, ent_hi, o_idx, o, ssl)

    s_h = _gru(ssl, ssh, esl, esh, rrl, rrh, W_ih_s, W_hh_s, b_ih_s, b_hh_s)
    o_h = _gru(osl, osh, eol, eoh, rrl, rrh, W_ih_o, W_hh_o, b_ih_o, b_hh_o)

    ob_pred, loss_ob = _logits_ce(esl, esh, s_h, rrl, rrh, W_sub.T,
                                  b_sub.reshape(-1, 1), o.reshape(1, -1))
    sub_pred, loss_sub = _logits_ce(eol, eoh, o_h, rrl, rrh, W_ob.T,
                                    b_ob.reshape(-1, 1), s.reshape(1, -1))

    loss = (loss_ob + loss_sub).reshape(())
    return (loss, sub_pred, ob_pred)
